# baseline (speedup 1.0000x reference)
import functools

import jax
import jax.numpy as jnp
from jax import lax
from jax.experimental import pallas as pl
from jax.experimental.pallas import tpu as pltpu
from jax.experimental.pallas import tpu_sc as plsc

_NUM_EMB = 1000000
_DIM = 32
_BATCH = 4096
_SEQ = 200
_NBT = _BATCH // 128


_CCH = 8192
_UG = (_NUM_EMB + _CCH - 1) // _CCH


def _untile_body(wt_ref, out_ref):
    x = wt_ref[...]
    out_ref[...] = jnp.transpose(x, (1, 0))


_untile = pl.pallas_call(
    _untile_body,
    grid=(_UG,),
    in_specs=[pl.BlockSpec((_DIM, _CCH), lambda j: (0, j))],
    out_specs=pl.BlockSpec((_CCH, _DIM), lambda j: (j, 0)),
    out_shape=jax.ShapeDtypeStruct((_NUM_EMB, _DIM), jnp.float32),
)


_info = plsc.get_sparse_core_info()
_NC, _NS = _info.num_cores, _info.num_subcores
_NW = _NC * _NS
_GS, _GBT = 8, 4
_SPG = _SEQ // _GS
_BTPG = _NBT // _GBT
_UNITS = _SPG * _BTPG

_mesh = plsc.VectorSubcoreMesh(core_axis_name="c", subcore_axis_name="s")


@functools.partial(
    pl.kernel,
    mesh=_mesh,
    out_type=jax.ShapeDtypeStruct((_SEQ, _NBT, 128, _DIM), jnp.float32),
    scratch_types=[
        pltpu.VMEM((_SPG, _BTPG, 128), jnp.int32),
        pltpu.VMEM((128, _DIM), jnp.float32),
        pltpu.VMEM((128, _DIM), jnp.float32),
        pltpu.SemaphoreType.DMA,
        pltpu.SemaphoreType.DMA,
        pltpu.SemaphoreType.DMA,
        pltpu.SemaphoreType.DMA,
    ],
    compiler_params=pltpu.CompilerParams(
        use_tc_tiling_on_sc=False, needs_layout_passes=False
    ),
)
def _gather_sc(table_hbm, idx_hbm, out_hbm, idx_v, rows0, rows1,
               semg0, semg1, sems0, sems1):
    wid = lax.axis_index("s") * _NC + lax.axis_index("c")
    gs = wid // _GBT
    gbt = wid % _GBT
    rows = (rows0, rows1)
    semg = (semg0, semg1)
    sems = (sems0, sems1)

    pltpu.sync_copy(
        idx_hbm.at[pl.ds(gs * _SPG, _SPG), pl.ds(gbt * _BTPG, _BTPG)], idx_v
    )

    def fire_g(u, buf):
        pltpu.async_copy(
            table_hbm.at[idx_v.at[u // _BTPG, u % _BTPG]], rows[buf], semg[buf]
        )

    def drain_g(buf):
        pltpu.make_async_copy(
            table_hbm.at[pl.ds(0, 128)], rows[buf], semg[buf]
        ).wait()

    def fire_s(u, buf):
        s = gs * _SPG + u // _BTPG
        bt = gbt * _BTPG + u % _BTPG
        pltpu.async_copy(rows[buf], out_hbm.at[s, bt], sems[buf])

    def drain_s(buf):
        pltpu.make_async_copy(rows[buf], out_hbm.at[0, 0], sems[buf]).wait()

    fire_g(0, 0)
    fire_g(1, 1)

    @pl.loop(0, _UNITS - 2, step=2)
    def _steady(u0):
        for d_ in range(2):
            u = u0 + d_
            buf = d_ % 2
            drain_g(buf)
            fire_s(u, buf)
            drain_s(buf)
            fire_g(u + 2, buf)

    for u, buf in ((_UNITS - 2, 0), (_UNITS - 1, 1)):
        drain_g(buf)
        fire_s(u, buf)
        drain_s(buf)


def _xpose_body(in_ref, out_ref):
    x = in_ref[0, 0]
    out_ref[0, ...] = jnp.transpose(x, (1, 0))


_xpose = pl.pallas_call(
    _xpose_body,
    grid=(_SEQ, _NBT),
    in_specs=[pl.BlockSpec((1, 1, 128, _DIM), lambda s, bt: (s, bt, 0, 0))],
    out_specs=pl.BlockSpec((1, _DIM, 128), lambda s, bt: (s, 0, bt)),
    out_shape=jax.ShapeDtypeStruct((_SEQ, _DIM, _BATCH), jnp.float32),
)


def kernel(input_ids, weight):
    table_lin = _untile(weight.T)
    idx3 = input_ids.T.reshape(_SEQ, _NBT, 128)
    inter = _gather_sc(table_lin, idx3)
    out = _xpose(inter)
    return out.transpose(2, 0, 1)


# speedup vs baseline: 2.8327x; 2.8327x over previous
import functools

import jax
import jax.numpy as jnp
from jax import lax
from jax.experimental import pallas as pl
from jax.experimental.pallas import tpu as pltpu
from jax.experimental.pallas import tpu_sc as plsc

_NUM_EMB = 1000000
_DIM = 32
_BATCH = 4096
_SEQ = 200
_NBT = _BATCH // 128


_CCH = 8192
_UG = (_NUM_EMB + _CCH - 1) // _CCH


def _untile_body(wt_ref, out_ref):
    x = wt_ref[...]
    out_ref[...] = jnp.transpose(x, (1, 0))


_untile = pl.pallas_call(
    _untile_body,
    grid=(_UG,),
    in_specs=[pl.BlockSpec((_DIM, _CCH), lambda j: (0, j))],
    out_specs=pl.BlockSpec((_CCH, _DIM), lambda j: (j, 0)),
    out_shape=jax.ShapeDtypeStruct((_NUM_EMB, _DIM), jnp.float32),
)


_info = plsc.get_sparse_core_info()
_NC, _NS = _info.num_cores, _info.num_subcores
_NW = _NC * _NS
_GS, _GBT = 8, 4
_SPG = _SEQ // _GS
_BTPG = _NBT // _GBT
_UNITS = _SPG * _BTPG

_mesh = plsc.VectorSubcoreMesh(core_axis_name="c", subcore_axis_name="s")


@functools.partial(
    pl.kernel,
    mesh=_mesh,
    out_type=jax.ShapeDtypeStruct((_SEQ, _NBT, 128, _DIM), jnp.float32),
    scratch_types=[
        pltpu.VMEM((_SPG, _BTPG, 128), jnp.int32),
        pltpu.VMEM((128, _DIM), jnp.float32),
        pltpu.VMEM((128, _DIM), jnp.float32),
        pltpu.SemaphoreType.DMA,
        pltpu.SemaphoreType.DMA,
        pltpu.SemaphoreType.DMA,
        pltpu.SemaphoreType.DMA,
    ],
    compiler_params=pltpu.CompilerParams(
        use_tc_tiling_on_sc=False, needs_layout_passes=False
    ),
)
def _gather_sc(table_hbm, idx_hbm, out_hbm, idx_v, rows0, rows1,
               semg0, semg1, sems0, sems1):
    wid = lax.axis_index("s") * _NC + lax.axis_index("c")
    gs = wid // _GBT
    gbt = wid % _GBT
    rows = (rows0, rows1)
    semg = (semg0, semg1)
    sems = (sems0, sems1)

    pltpu.sync_copy(
        idx_hbm.at[pl.ds(gs * _SPG, _SPG), pl.ds(gbt * _BTPG, _BTPG)], idx_v
    )

    def fire_g(u, buf):
        pltpu.async_copy(
            table_hbm.at[idx_v.at[u // _BTPG, u % _BTPG]], rows[buf], semg[buf]
        )

    def drain_g(buf):
        pltpu.make_async_copy(
            table_hbm.at[pl.ds(0, 128)], rows[buf], semg[buf]
        ).wait()

    def fire_s(u, buf):
        s = gs * _SPG + u // _BTPG
        bt = gbt * _BTPG + u % _BTPG
        pltpu.async_copy(rows[buf], out_hbm.at[s, bt], sems[buf])

    def drain_s(buf):
        pltpu.make_async_copy(rows[buf], out_hbm.at[0, 0], sems[buf]).wait()

    fire_g(0, 0)
    fire_g(1, 1)

    @pl.loop(0, _UNITS - 2, step=2)
    def _steady(u0):
        for d_ in range(2):
            u = u0 + d_
            buf = d_ % 2
            drain_g(buf)
            fire_s(u, buf)
            drain_s(buf)
            fire_g(u + 2, buf)

    for u, buf in ((_UNITS - 2, 0), (_UNITS - 1, 1)):
        drain_g(buf)
        fire_s(u, buf)
        drain_s(buf)


_XBT = 8


def _xpose_body(in_ref, out_ref):
    for k in range(_XBT):
        x = in_ref[0, k]
        out_ref[0, :, k * 128:(k + 1) * 128] = jnp.transpose(x, (1, 0))


_xpose = pl.pallas_call(
    _xpose_body,
    grid=(_SEQ, _NBT // _XBT),
    in_specs=[pl.BlockSpec((1, _XBT, 128, _DIM), lambda s, g: (s, g, 0, 0))],
    out_specs=pl.BlockSpec((1, _DIM, _XBT * 128), lambda s, g: (s, 0, g)),
    out_shape=jax.ShapeDtypeStruct((_SEQ, _DIM, _BATCH), jnp.float32),
)


def kernel(input_ids, weight):
    table_lin = _untile(weight.T)
    idx3 = input_ids.T.reshape(_SEQ, _NBT, 128)
    inter = _gather_sc(table_lin, idx3)
    out = _xpose(inter)
    return out.transpose(2, 0, 1)
